# baseline (speedup 1.0000x reference)
import functools

import jax
import jax.numpy as jnp
from jax import lax
from jax.experimental import pallas as pl
from jax.experimental.pallas import tpu as pltpu
from jax.experimental.pallas import tpu_sc as plsc

_BATCH, _ROW, _COL = 16, 4096, 256
_NS = 64


def _matmul_body(param_ref, x_ref, out_ref, m_ref):
    b = pl.program_id(0)

    @pl.when(b == 0)
    def _build_mask():
        idx = jax.lax.broadcasted_iota(jnp.int32, (_NS, _ROW), 1)
        s0 = param_ref[:, 0:1]
        s1 = param_ref[:, 1:2]
        mask = (idx >= s0) & (idx < s1)
        m_ref[...] = mask.astype(jnp.bfloat16)

    for i in range(_BPB):
        x = x_ref[i].astype(jnp.bfloat16)
        out_ref[i] = jax.lax.dot(
            m_ref[...], x, preferred_element_type=jnp.float32
        )


_BPB = 2


_SC_C = 32
_SC_NW = 32
_SC_TPW = (_NS * _BATCH) // _SC_NW
_SC_NJ = _COL // 16


def _sc_body(x_hbm, param_hbm, out_hbm, param_v, param_s, chunk_v, res_v, sem):
    wid = lax.axis_index("s") * 2 + lax.axis_index("c")
    pltpu.sync_copy(param_hbm, param_v)
    b = wid // 2
    ihalf = wid % 2
    for grp in range(_NS // 16):
        off16 = pl.multiple_of(ihalf * _NS + grp * 16, 16)
        pv = param_v[pl.ds(off16, 16)]
        for lane in range(16):
            param_s[grp * 16 + lane] = pv[lane]

    def run_task(tau, carry):
        s0 = param_s[2 * tau]
        s1 = param_s[2 * tau + 1]
        a0 = pl.multiple_of((s0 // 8) * 8, 8)
        nc = (s1 - a0 + _SC_C - 1) // _SC_C

        for j in range(_SC_NJ):
            res_v[tau, pl.ds(16 * j, 16)] = jnp.zeros((16,), jnp.float32)

        def chunk_step(c, inner):
            lo = a0 + c * _SC_C
            start = pl.multiple_of(jnp.minimum(lo, _ROW - _SC_C), 8)
            pltpu.sync_copy(x_hbm.at[b, pl.ds(start, _SC_C), :], chunk_v)
            for r in range(_SC_C):
                g = start + r
                valid = (g >= s0) & (g >= lo) & (g < s1)
                w = jnp.where(valid, 1.0, 0.0).astype(jnp.float32)
                for j in range(_SC_NJ):
                    v = chunk_v[r, pl.ds(16 * j, 16)]
                    plsc.addupdate(res_v.at[tau, pl.ds(16 * j, 16)], v * w)
            return inner

        lax.fori_loop(0, nc, chunk_step, 0)
        return carry

    lax.fori_loop(0, _NS // 2, run_task, 0)
    pltpu.sync_copy(
        res_v, out_hbm.at[b, pl.ds(ihalf * (_NS // 2), _NS // 2), :]
    )


def _sc_kernel(input, slice_param):
    mesh = plsc.VectorSubcoreMesh(core_axis_name="c", subcore_axis_name="s")
    out = pl.kernel(
        _sc_body,
        mesh=mesh,
        out_type=jax.ShapeDtypeStruct((_BATCH, _NS, _COL), jnp.float32),
        scratch_types=[
            pltpu.VMEM((2 * _NS,), jnp.int32),
            pltpu.SMEM((_NS,), jnp.int32),
            pltpu.VMEM((_SC_C, _COL), jnp.float32),
            pltpu.VMEM((_NS // 2, _COL), jnp.float32),
            pltpu.SemaphoreType.DMA,
        ],
    )(input, slice_param.reshape(-1))
    return out.reshape(_BATCH, _NS * _COL)


def _tc_kernel(input, slice_param):
    out = pl.pallas_call(
        _matmul_body,
        grid=(_BATCH // _BPB,),
        in_specs=[
            pl.BlockSpec((_NS, 2), lambda b: (0, 0)),
            pl.BlockSpec((_BPB, _ROW, _COL), lambda b: (b, 0, 0)),
        ],
        out_specs=pl.BlockSpec((_BPB, _NS, _COL), lambda b: (b, 0, 0)),
        out_shape=jax.ShapeDtypeStruct((_BATCH, _NS, _COL), jnp.float32),
        scratch_shapes=[pltpu.VMEM((_NS, _ROW), jnp.bfloat16)],
    )(slice_param, input)
    return out.reshape(_BATCH, _NS * _COL)


kernel = _sc_kernel


# speedup vs baseline: 4.4079x; 4.4079x over previous
import functools

import jax
import jax.numpy as jnp
from jax import lax
from jax.experimental import pallas as pl
from jax.experimental.pallas import tpu as pltpu
from jax.experimental.pallas import tpu_sc as plsc

_BATCH, _ROW, _COL = 16, 4096, 256
_NS = 64


def _matmul_body(param_ref, x_ref, out_ref, m_ref):
    b = pl.program_id(0)

    @pl.when(b == 0)
    def _build_mask():
        idx = jax.lax.broadcasted_iota(jnp.int32, (_NS, _ROW), 1)
        s0 = param_ref[:, 0:1]
        s1 = param_ref[:, 1:2]
        mask = (idx >= s0) & (idx < s1)
        m_ref[...] = mask.astype(jnp.bfloat16)

    for i in range(_BPB):
        x = x_ref[i].astype(jnp.bfloat16)
        out_ref[i] = jax.lax.dot(
            m_ref[...], x, preferred_element_type=jnp.float32
        )


_BPB = 2


_SC_C = 64
_SC_NW = 32
_SC_TPW = (_NS * _BATCH) // _SC_NW
_SC_NJ = _COL // 16


def _sc_body(
    x_hbm, param_hbm, out_hbm, param_v, param_s, chunk_a, chunk_b, res_v,
    sem_a, sem_b,
):
    wid = lax.axis_index("s") * 2 + lax.axis_index("c")
    pltpu.sync_copy(param_hbm, param_v)
    b = wid // 2
    ihalf = wid % 2
    for grp in range(_NS // 16):
        off16 = pl.multiple_of(ihalf * _NS + grp * 16, 16)
        pv = param_v[pl.ds(off16, 16)]
        for lane in range(16):
            param_s[grp * 16 + lane] = pv[lane]

    def _chunk_start(a0, c):
        lo = a0 + c * _SC_C
        return lo, pl.multiple_of(jnp.minimum(lo, _ROW - _SC_C), 8)

    def _accumulate(buf, lo, start, s0, s1, accs):
        def row_step(r, a):
            g = start + r
            valid = (g >= s0) & (g >= lo) & (g < s1)
            w = jnp.where(valid, 1.0, 0.0).astype(jnp.float32)
            new = [a[j] + buf[r, pl.ds(16 * j, 16)] * w for j in range(_SC_NJ)]
            return tuple(new)
        return lax.fori_loop(0, _SC_C, row_step, accs)

    def run_task(tau, carry):
        s0 = param_s[2 * tau]
        s1 = param_s[2 * tau + 1]
        a0 = pl.multiple_of((s0 // 8) * 8, 8)
        nc = (s1 - a0 + _SC_C - 1) // _SC_C
        npairs = (nc + 1) // 2

        accs = tuple(jnp.zeros((16,), jnp.float32) for _ in range(_SC_NJ))
        lo0, st0 = _chunk_start(a0, 0)
        cpA = pltpu.async_copy(x_hbm.at[b, pl.ds(st0, _SC_C), :], chunk_a, sem_a)

        def pair_step(p, accs):
            loA, stA = _chunk_start(a0, 2 * p)
            loB, stB = _chunk_start(a0, 2 * p + 1)
            pltpu.make_async_copy(
                x_hbm.at[b, pl.ds(stA, _SC_C), :], chunk_a, sem_a
            ).wait()
            cpB = pltpu.async_copy(
                x_hbm.at[b, pl.ds(stB, _SC_C), :], chunk_b, sem_b
            )
            accs = _accumulate(chunk_a, loA, stA, s0, s1, accs)
            loN, stN = _chunk_start(a0, 2 * p + 2)
            cpB.wait()
            pltpu.async_copy(x_hbm.at[b, pl.ds(stN, _SC_C), :], chunk_a, sem_a)
            accs = _accumulate(chunk_b, loB, stB, s0, s1, accs)
            return accs

        accs = lax.fori_loop(0, npairs, pair_step, accs)
        lo0, st0 = _chunk_start(a0, 0)
        pltpu.make_async_copy(
            x_hbm.at[b, pl.ds(st0, _SC_C), :], chunk_a, sem_a
        ).wait()
        for j in range(_SC_NJ):
            res_v[tau, pl.ds(16 * j, 16)] = accs[j]
        return carry

    lax.fori_loop(0, _NS // 2, run_task, 0)
    pltpu.sync_copy(
        res_v, out_hbm.at[b, pl.ds(ihalf * (_NS // 2), _NS // 2), :]
    )


def _sc_kernel(input, slice_param):
    mesh = plsc.VectorSubcoreMesh(core_axis_name="c", subcore_axis_name="s")
    out = pl.kernel(
        _sc_body,
        mesh=mesh,
        out_type=jax.ShapeDtypeStruct((_BATCH, _NS, _COL), jnp.float32),
        scratch_types=[
            pltpu.VMEM((2 * _NS,), jnp.int32),
            pltpu.SMEM((_NS,), jnp.int32),
            pltpu.VMEM((_SC_C, _COL), jnp.float32),
            pltpu.VMEM((_SC_C, _COL), jnp.float32),
            pltpu.VMEM((_NS // 2, _COL), jnp.float32),
            pltpu.SemaphoreType.DMA,
            pltpu.SemaphoreType.DMA,
        ],
    )(input, slice_param.reshape(-1))
    return out.reshape(_BATCH, _NS * _COL)


def _tc_kernel(input, slice_param):
    out = pl.pallas_call(
        _matmul_body,
        grid=(_BATCH // _BPB,),
        in_specs=[
            pl.BlockSpec((_NS, 2), lambda b: (0, 0)),
            pl.BlockSpec((_BPB, _ROW, _COL), lambda b: (b, 0, 0)),
        ],
        out_specs=pl.BlockSpec((_BPB, _NS, _COL), lambda b: (b, 0, 0)),
        out_shape=jax.ShapeDtypeStruct((_BATCH, _NS, _COL), jnp.float32),
        scratch_shapes=[pltpu.VMEM((_NS, _ROW), jnp.bfloat16)],
    )(slice_param, input)
    return out.reshape(_BATCH, _NS * _COL)


kernel = _sc_kernel


# speedup vs baseline: 4.4361x; 1.0064x over previous
import functools

import jax
import jax.numpy as jnp
from jax import lax
from jax.experimental import pallas as pl
from jax.experimental.pallas import tpu as pltpu
from jax.experimental.pallas import tpu_sc as plsc

_BATCH, _ROW, _COL = 16, 4096, 256
_NS = 64


def _matmul_body(param_ref, x_ref, out_ref, m_ref):
    b = pl.program_id(0)

    @pl.when(b == 0)
    def _build_mask():
        idx = jax.lax.broadcasted_iota(jnp.int32, (_NS, _ROW), 1)
        s0 = param_ref[:, 0:1]
        s1 = param_ref[:, 1:2]
        mask = (idx >= s0) & (idx < s1)
        m_ref[...] = mask.astype(jnp.bfloat16)

    for i in range(_BPB):
        x = x_ref[i].astype(jnp.bfloat16)
        out_ref[i] = jax.lax.dot(
            m_ref[...], x, preferred_element_type=jnp.float32
        )


_BPB = 2


_SC_C = 64
_SC_NW = 32
_SC_TPW = (_NS * _BATCH) // _SC_NW
_SC_NJ = _COL // 16


def _sc_body(
    x_hbm, param_hbm, out_hbm, param_v, param_s, chunk_a, chunk_b, res_v,
    sem_a, sem_b,
):
    wid = lax.axis_index("s") * 2 + lax.axis_index("c")
    pltpu.sync_copy(param_hbm, param_v)
    b = wid // 2
    ihalf = wid % 2
    for grp in range(_NS // 16):
        off16 = pl.multiple_of(ihalf * _NS + grp * 16, 16)
        pv = param_v[pl.ds(off16, 16)]
        for lane in range(16):
            param_s[grp * 16 + lane] = pv[lane]

    def _chunk_start(a0, c):
        lo = a0 + c * _SC_C
        return lo, pl.multiple_of(jnp.minimum(lo, _ROW - _SC_C), 8)

    def _accumulate(buf, lo, start, s0, s1, accs):
        rlo = jnp.maximum(jnp.maximum(s0, lo) - start, 0)
        rhi = jnp.minimum(s1 - start, _SC_C)

        def row_step(r, a):
            return tuple(
                a[j] + buf[r, pl.ds(16 * j, 16)] for j in range(_SC_NJ)
            )

        return lax.fori_loop(rlo, rhi, row_step, accs)

    def run_task(tau, carry):
        s0 = param_s[2 * tau]
        s1 = param_s[2 * tau + 1]
        a0 = pl.multiple_of((s0 // 8) * 8, 8)
        nc = (s1 - a0 + _SC_C - 1) // _SC_C
        npairs = (nc + 1) // 2

        accs = tuple(jnp.zeros((16,), jnp.float32) for _ in range(_SC_NJ))
        lo0, st0 = _chunk_start(a0, 0)
        cpA = pltpu.async_copy(x_hbm.at[b, pl.ds(st0, _SC_C), :], chunk_a, sem_a)

        def pair_step(p, accs):
            loA, stA = _chunk_start(a0, 2 * p)
            loB, stB = _chunk_start(a0, 2 * p + 1)
            pltpu.make_async_copy(
                x_hbm.at[b, pl.ds(stA, _SC_C), :], chunk_a, sem_a
            ).wait()
            cpB = pltpu.async_copy(
                x_hbm.at[b, pl.ds(stB, _SC_C), :], chunk_b, sem_b
            )
            accs = _accumulate(chunk_a, loA, stA, s0, s1, accs)
            loN, stN = _chunk_start(a0, 2 * p + 2)
            cpB.wait()
            pltpu.async_copy(x_hbm.at[b, pl.ds(stN, _SC_C), :], chunk_a, sem_a)
            accs = _accumulate(chunk_b, loB, stB, s0, s1, accs)
            return accs

        accs = lax.fori_loop(0, npairs, pair_step, accs)
        lo0, st0 = _chunk_start(a0, 0)
        pltpu.make_async_copy(
            x_hbm.at[b, pl.ds(st0, _SC_C), :], chunk_a, sem_a
        ).wait()
        for j in range(_SC_NJ):
            res_v[tau, pl.ds(16 * j, 16)] = accs[j]
        return carry

    lax.fori_loop(0, _NS // 2, run_task, 0)
    pltpu.sync_copy(
        res_v, out_hbm.at[b, pl.ds(ihalf * (_NS // 2), _NS // 2), :]
    )


def _sc_kernel(input, slice_param):
    mesh = plsc.VectorSubcoreMesh(core_axis_name="c", subcore_axis_name="s")
    out = pl.kernel(
        _sc_body,
        mesh=mesh,
        out_type=jax.ShapeDtypeStruct((_BATCH, _NS, _COL), jnp.float32),
        scratch_types=[
            pltpu.VMEM((2 * _NS,), jnp.int32),
            pltpu.SMEM((_NS,), jnp.int32),
            pltpu.VMEM((_SC_C, _COL), jnp.float32),
            pltpu.VMEM((_SC_C, _COL), jnp.float32),
            pltpu.VMEM((_NS // 2, _COL), jnp.float32),
            pltpu.SemaphoreType.DMA,
            pltpu.SemaphoreType.DMA,
        ],
    )(input, slice_param.reshape(-1))
    return out.reshape(_BATCH, _NS * _COL)


def _tc_kernel(input, slice_param):
    out = pl.pallas_call(
        _matmul_body,
        grid=(_BATCH // _BPB,),
        in_specs=[
            pl.BlockSpec((_NS, 2), lambda b: (0, 0)),
            pl.BlockSpec((_BPB, _ROW, _COL), lambda b: (b, 0, 0)),
        ],
        out_specs=pl.BlockSpec((_BPB, _NS, _COL), lambda b: (b, 0, 0)),
        out_shape=jax.ShapeDtypeStruct((_BATCH, _NS, _COL), jnp.float32),
        scratch_shapes=[pltpu.VMEM((_NS, _ROW), jnp.bfloat16)],
    )(slice_param, input)
    return out.reshape(_BATCH, _NS * _COL)


kernel = _sc_kernel


# speedup vs baseline: 4.6183x; 1.0411x over previous
import functools

import jax
import jax.numpy as jnp
from jax import lax
from jax.experimental import pallas as pl
from jax.experimental.pallas import tpu as pltpu
from jax.experimental.pallas import tpu_sc as plsc

_BATCH, _ROW, _COL = 16, 4096, 256
_NS = 64


def _matmul_body(param_ref, x_ref, out_ref, m_ref):
    b = pl.program_id(0)

    @pl.when(b == 0)
    def _build_mask():
        idx = jax.lax.broadcasted_iota(jnp.int32, (_NS, _ROW), 1)
        s0 = param_ref[:, 0:1]
        s1 = param_ref[:, 1:2]
        mask = (idx >= s0) & (idx < s1)
        m_ref[...] = mask.astype(jnp.bfloat16)

    for i in range(_BPB):
        x = x_ref[i].astype(jnp.bfloat16)
        out_ref[i] = jax.lax.dot(
            m_ref[...], x, preferred_element_type=jnp.float32
        )


_BPB = 2


_SC_C = 128
_SC_NW = 32
_SC_TPW = (_NS * _BATCH) // _SC_NW
_SC_NJ = _COL // 16


def _sc_body(
    x_hbm, param_hbm, out_hbm, param_v, param_s, chunk_a, chunk_b, res_v,
    sem_a, sem_b,
):
    wid = lax.axis_index("s") * 2 + lax.axis_index("c")
    pltpu.sync_copy(param_hbm, param_v)
    b = wid // 2
    ihalf = wid % 2
    for grp in range(_NS // 16):
        off16 = pl.multiple_of(ihalf * _NS + grp * 16, 16)
        pv = param_v[pl.ds(off16, 16)]
        for lane in range(16):
            param_s[grp * 16 + lane] = pv[lane]

    def _chunk_start(a0, c):
        lo = a0 + c * _SC_C
        return lo, pl.multiple_of(jnp.minimum(lo, _ROW - _SC_C), 8)

    def _accumulate(buf, lo, start, s0, s1, accs):
        rlo = jnp.maximum(jnp.maximum(s0, lo) - start, 0)
        rhi = jnp.minimum(s1 - start, _SC_C)

        def row_step(r, a):
            return tuple(
                a[j] + buf[r, pl.ds(16 * j, 16)] for j in range(_SC_NJ)
            )

        return lax.fori_loop(rlo, rhi, row_step, accs)

    def run_task(tau, carry):
        s0 = param_s[2 * tau]
        s1 = param_s[2 * tau + 1]
        a0 = pl.multiple_of((s0 // 8) * 8, 8)
        nc = (s1 - a0 + _SC_C - 1) // _SC_C
        npairs = (nc + 1) // 2

        accs = tuple(jnp.zeros((16,), jnp.float32) for _ in range(_SC_NJ))
        lo0, st0 = _chunk_start(a0, 0)
        cpA = pltpu.async_copy(x_hbm.at[b, pl.ds(st0, _SC_C), :], chunk_a, sem_a)

        def pair_step(p, accs):
            loA, stA = _chunk_start(a0, 2 * p)
            loB, stB = _chunk_start(a0, 2 * p + 1)
            pltpu.make_async_copy(
                x_hbm.at[b, pl.ds(stA, _SC_C), :], chunk_a, sem_a
            ).wait()
            cpB = pltpu.async_copy(
                x_hbm.at[b, pl.ds(stB, _SC_C), :], chunk_b, sem_b
            )
            accs = _accumulate(chunk_a, loA, stA, s0, s1, accs)
            loN, stN = _chunk_start(a0, 2 * p + 2)
            cpB.wait()
            pltpu.async_copy(x_hbm.at[b, pl.ds(stN, _SC_C), :], chunk_a, sem_a)
            accs = _accumulate(chunk_b, loB, stB, s0, s1, accs)
            return accs

        accs = lax.fori_loop(0, npairs, pair_step, accs)
        lo0, st0 = _chunk_start(a0, 0)
        pltpu.make_async_copy(
            x_hbm.at[b, pl.ds(st0, _SC_C), :], chunk_a, sem_a
        ).wait()
        for j in range(_SC_NJ):
            res_v[tau, pl.ds(16 * j, 16)] = accs[j]
        return carry

    lax.fori_loop(0, _NS // 2, run_task, 0)
    pltpu.sync_copy(
        res_v, out_hbm.at[b, pl.ds(ihalf * (_NS // 2), _NS // 2), :]
    )


def _sc_kernel(input, slice_param):
    mesh = plsc.VectorSubcoreMesh(core_axis_name="c", subcore_axis_name="s")
    out = pl.kernel(
        _sc_body,
        mesh=mesh,
        out_type=jax.ShapeDtypeStruct((_BATCH, _NS, _COL), jnp.float32),
        scratch_types=[
            pltpu.VMEM((2 * _NS,), jnp.int32),
            pltpu.SMEM((_NS,), jnp.int32),
            pltpu.VMEM((_SC_C, _COL), jnp.float32),
            pltpu.VMEM((_SC_C, _COL), jnp.float32),
            pltpu.VMEM((_NS // 2, _COL), jnp.float32),
            pltpu.SemaphoreType.DMA,
            pltpu.SemaphoreType.DMA,
        ],
    )(input, slice_param.reshape(-1))
    return out.reshape(_BATCH, _NS * _COL)


def _tc_kernel(input, slice_param):
    out = pl.pallas_call(
        _matmul_body,
        grid=(_BATCH // _BPB,),
        in_specs=[
            pl.BlockSpec((_NS, 2), lambda b: (0, 0)),
            pl.BlockSpec((_BPB, _ROW, _COL), lambda b: (b, 0, 0)),
        ],
        out_specs=pl.BlockSpec((_BPB, _NS, _COL), lambda b: (b, 0, 0)),
        out_shape=jax.ShapeDtypeStruct((_BATCH, _NS, _COL), jnp.float32),
        scratch_shapes=[pltpu.VMEM((_NS, _ROW), jnp.bfloat16)],
    )(slice_param, input)
    return out.reshape(_BATCH, _NS * _COL)


kernel = _sc_kernel


# speedup vs baseline: 52.3188x; 11.3287x over previous
import functools

import jax
import jax.numpy as jnp
from jax import lax
from jax.experimental import pallas as pl
from jax.experimental.pallas import tpu as pltpu
from jax.experimental.pallas import tpu_sc as plsc

_BATCH, _ROW, _COL = 16, 4096, 256
_NS = 64


def _matmul_body(param_ref, x_ref, out_ref, m_ref):
    b = pl.program_id(0)

    @pl.when(b == 0)
    def _build_mask():
        idx = jax.lax.broadcasted_iota(jnp.int32, (_NS, _ROW), 1)
        s0 = param_ref[:, 0:1]
        s1 = param_ref[:, 1:2]
        mask = (idx >= s0) & (idx < s1)
        m_ref[...] = mask.astype(jnp.bfloat16)

    for i in range(_BPB):
        x = x_ref[i].astype(jnp.bfloat16)
        out_ref[i] = jax.lax.dot(
            m_ref[...], x, preferred_element_type=jnp.float32
        )


_BPB = 2


_SC_C = 128
_SC_NW = 32
_SC_TPW = (_NS * _BATCH) // _SC_NW
_SC_NJ = _COL // 16


def _sc_body(
    x_hbm, param_hbm, out_hbm, param_v, param_s, chunk_a, chunk_b, res_v,
    sem_a, sem_b,
):
    wid = lax.axis_index("s") * 2 + lax.axis_index("c")
    pltpu.sync_copy(param_hbm, param_v)
    b = wid // 2
    ihalf = wid % 2
    for grp in range(_NS // 16):
        off16 = pl.multiple_of(ihalf * _NS + grp * 16, 16)
        pv = param_v[pl.ds(off16, 16)]
        for lane in range(16):
            param_s[grp * 16 + lane] = pv[lane]

    def _chunk_start(a0, c):
        lo = a0 + c * _SC_C
        return lo, pl.multiple_of(jnp.minimum(lo, _ROW - _SC_C), 8)

    def _accumulate(buf, lo, start, s0, s1, accs):
        rlo = jnp.maximum(jnp.maximum(s0, lo) - start, 0)
        rhi = jnp.minimum(s1 - start, _SC_C)

        def row_step(r, a):
            return tuple(
                a[j] + buf[r, pl.ds(16 * j, 16)] for j in range(_SC_NJ)
            )

        return lax.fori_loop(rlo, rhi, row_step, accs)

    def run_task(tau, carry):
        s0 = param_s[2 * tau]
        s1 = param_s[2 * tau + 1]
        a0 = pl.multiple_of((s0 // 8) * 8, 8)
        nc = (s1 - a0 + _SC_C - 1) // _SC_C
        npairs = (nc + 1) // 2

        accs = tuple(jnp.zeros((16,), jnp.float32) for _ in range(_SC_NJ))
        lo0, st0 = _chunk_start(a0, 0)
        cpA = pltpu.async_copy(x_hbm.at[b, pl.ds(st0, _SC_C), :], chunk_a, sem_a)

        def pair_step(p, accs):
            loA, stA = _chunk_start(a0, 2 * p)
            loB, stB = _chunk_start(a0, 2 * p + 1)
            pltpu.make_async_copy(
                x_hbm.at[b, pl.ds(stA, _SC_C), :], chunk_a, sem_a
            ).wait()
            cpB = pltpu.async_copy(
                x_hbm.at[b, pl.ds(stB, _SC_C), :], chunk_b, sem_b
            )
            accs = _accumulate(chunk_a, loA, stA, s0, s1, accs)
            loN, stN = _chunk_start(a0, 2 * p + 2)
            cpB.wait()
            pltpu.async_copy(x_hbm.at[b, pl.ds(stN, _SC_C), :], chunk_a, sem_a)
            accs = _accumulate(chunk_b, loB, stB, s0, s1, accs)
            return accs

        accs = lax.fori_loop(0, npairs, pair_step, accs)
        lo0, st0 = _chunk_start(a0, 0)
        pltpu.make_async_copy(
            x_hbm.at[b, pl.ds(st0, _SC_C), :], chunk_a, sem_a
        ).wait()
        for j in range(_SC_NJ):
            res_v[tau, pl.ds(16 * j, 16)] = accs[j]
        return carry

    lax.fori_loop(0, _NS // 2, run_task, 0)
    pltpu.sync_copy(
        res_v, out_hbm.at[b, pl.ds(ihalf * (_NS // 2), _NS // 2), :]
    )


def _sc_kernel(input, slice_param):
    mesh = plsc.VectorSubcoreMesh(core_axis_name="c", subcore_axis_name="s")
    out = pl.kernel(
        _sc_body,
        mesh=mesh,
        out_type=jax.ShapeDtypeStruct((_BATCH, _NS, _COL), jnp.float32),
        scratch_types=[
            pltpu.VMEM((2 * _NS,), jnp.int32),
            pltpu.SMEM((_NS,), jnp.int32),
            pltpu.VMEM((_SC_C, _COL), jnp.float32),
            pltpu.VMEM((_SC_C, _COL), jnp.float32),
            pltpu.VMEM((_NS // 2, _COL), jnp.float32),
            pltpu.SemaphoreType.DMA,
            pltpu.SemaphoreType.DMA,
        ],
    )(input, slice_param.reshape(-1))
    return out.reshape(_BATCH, _NS * _COL)


def _tc_kernel(input, slice_param):
    out = pl.pallas_call(
        _matmul_body,
        grid=(_BATCH // _BPB,),
        in_specs=[
            pl.BlockSpec((_NS, 2), lambda b: (0, 0)),
            pl.BlockSpec((_BPB, _ROW, _COL), lambda b: (b, 0, 0)),
        ],
        out_specs=pl.BlockSpec((_BPB, _NS, _COL), lambda b: (b, 0, 0)),
        out_shape=jax.ShapeDtypeStruct((_BATCH, _NS, _COL), jnp.float32),
        scratch_shapes=[pltpu.VMEM((_NS, _ROW), jnp.bfloat16)],
    )(slice_param, input)
    return out.reshape(_BATCH, _NS * _COL)


kernel = _tc_kernel


# speedup vs baseline: 52.6549x; 1.0064x over previous
import functools

import jax
import jax.numpy as jnp
from jax import lax
from jax.experimental import pallas as pl
from jax.experimental.pallas import tpu as pltpu
from jax.experimental.pallas import tpu_sc as plsc

_BATCH, _ROW, _COL = 16, 4096, 256
_NS = 64


def _matmul_body(param_ref, x_ref, out_ref, m_ref):
    b = pl.program_id(0)

    @pl.when(b == 0)
    def _build_mask():
        idx = jax.lax.broadcasted_iota(jnp.int32, (_NS, _ROW), 1)
        s0 = param_ref[:, 0:1]
        s1 = param_ref[:, 1:2]
        mask = (idx >= s0) & (idx < s1)
        m_ref[...] = mask.astype(jnp.bfloat16)

    for i in range(_BPB):
        x = x_ref[i].astype(jnp.bfloat16)
        out_ref[i] = jax.lax.dot(
            m_ref[...], x, preferred_element_type=jnp.float32
        )


_BPB = 2


_SC_C = 128
_SC_NW = 32
_SC_TPW = (_NS * _BATCH) // _SC_NW
_SC_NJ = _COL // 16


def _sc_body(
    x_hbm, param_hbm, out_hbm, param_v, param_s, chunk_a, chunk_b, res_v,
    sem_a, sem_b,
):
    wid = lax.axis_index("s") * 2 + lax.axis_index("c")
    pltpu.sync_copy(param_hbm, param_v)
    b = wid // 2
    ihalf = wid % 2
    for grp in range(_NS // 16):
        off16 = pl.multiple_of(ihalf * _NS + grp * 16, 16)
        pv = param_v[pl.ds(off16, 16)]
        for lane in range(16):
            param_s[grp * 16 + lane] = pv[lane]

    def _chunk_start(a0, c):
        lo = a0 + c * _SC_C
        return lo, pl.multiple_of(jnp.minimum(lo, _ROW - _SC_C), 8)

    def _accumulate(buf, lo, start, s0, s1, accs):
        rlo = jnp.maximum(jnp.maximum(s0, lo) - start, 0)
        rhi = jnp.minimum(s1 - start, _SC_C)

        def row_step(r, a):
            return tuple(
                a[j] + buf[r, pl.ds(16 * j, 16)] for j in range(_SC_NJ)
            )

        return lax.fori_loop(rlo, rhi, row_step, accs)

    def run_task(tau, carry):
        s0 = param_s[2 * tau]
        s1 = param_s[2 * tau + 1]
        a0 = pl.multiple_of((s0 // 8) * 8, 8)
        nc = (s1 - a0 + _SC_C - 1) // _SC_C
        npairs = (nc + 1) // 2

        accs = tuple(jnp.zeros((16,), jnp.float32) for _ in range(_SC_NJ))
        lo0, st0 = _chunk_start(a0, 0)
        cpA = pltpu.async_copy(x_hbm.at[b, pl.ds(st0, _SC_C), :], chunk_a, sem_a)

        def pair_step(p, accs):
            loA, stA = _chunk_start(a0, 2 * p)
            loB, stB = _chunk_start(a0, 2 * p + 1)
            pltpu.make_async_copy(
                x_hbm.at[b, pl.ds(stA, _SC_C), :], chunk_a, sem_a
            ).wait()
            cpB = pltpu.async_copy(
                x_hbm.at[b, pl.ds(stB, _SC_C), :], chunk_b, sem_b
            )
            accs = _accumulate(chunk_a, loA, stA, s0, s1, accs)
            loN, stN = _chunk_start(a0, 2 * p + 2)
            cpB.wait()
            pltpu.async_copy(x_hbm.at[b, pl.ds(stN, _SC_C), :], chunk_a, sem_a)
            accs = _accumulate(chunk_b, loB, stB, s0, s1, accs)
            return accs

        accs = lax.fori_loop(0, npairs, pair_step, accs)
        lo0, st0 = _chunk_start(a0, 0)
        pltpu.make_async_copy(
            x_hbm.at[b, pl.ds(st0, _SC_C), :], chunk_a, sem_a
        ).wait()
        for j in range(_SC_NJ):
            res_v[tau, pl.ds(16 * j, 16)] = accs[j]
        return carry

    lax.fori_loop(0, _NS // 2, run_task, 0)
    pltpu.sync_copy(
        res_v, out_hbm.at[b, pl.ds(ihalf * (_NS // 2), _NS // 2), :]
    )


def _sc_kernel(input, slice_param):
    mesh = plsc.VectorSubcoreMesh(core_axis_name="c", subcore_axis_name="s")
    out = pl.kernel(
        _sc_body,
        mesh=mesh,
        out_type=jax.ShapeDtypeStruct((_BATCH, _NS, _COL), jnp.float32),
        scratch_types=[
            pltpu.VMEM((2 * _NS,), jnp.int32),
            pltpu.SMEM((_NS,), jnp.int32),
            pltpu.VMEM((_SC_C, _COL), jnp.float32),
            pltpu.VMEM((_SC_C, _COL), jnp.float32),
            pltpu.VMEM((_NS // 2, _COL), jnp.float32),
            pltpu.SemaphoreType.DMA,
            pltpu.SemaphoreType.DMA,
        ],
    )(input, slice_param.reshape(-1))
    return out.reshape(_BATCH, _NS * _COL)


def _tc_body2d(param_ref, x_ref, out_ref, m_ref):
    b = pl.program_id(0)

    @pl.when(b == 0)
    def _build_mask():
        idx = jax.lax.broadcasted_iota(jnp.int32, (_NS, _ROW), 1)
        s0 = param_ref[:, 0:1]
        s1 = param_ref[:, 1:2]
        mask = (idx >= s0) & (idx < s1)
        m_ref[...] = mask.astype(jnp.bfloat16)

    for i in range(_BPB):
        x = x_ref[pl.ds(i * _ROW, _ROW), :].astype(jnp.bfloat16)
        out_ref[i] = jax.lax.dot(
            m_ref[...], x, preferred_element_type=jnp.float32
        )


def _tc_kernel(input, slice_param):
    out = pl.pallas_call(
        _tc_body2d,
        grid=(_BATCH // _BPB,),
        in_specs=[
            pl.BlockSpec((_NS, 2), lambda b: (0, 0)),
            pl.BlockSpec((_BPB * _ROW, _COL), lambda b: (b, 0)),
        ],
        out_specs=pl.BlockSpec((_BPB, _NS, _COL), lambda b: (b, 0, 0)),
        out_shape=jax.ShapeDtypeStruct((_BATCH, _NS, _COL), jnp.float32),
        scratch_shapes=[pltpu.VMEM((_NS, _ROW), jnp.bfloat16)],
    )(slice_param, input.reshape(_BATCH * _ROW, _COL))
    return out.reshape(_BATCH, _NS * _COL)


kernel = _tc_kernel
